# Initial kernel scaffold; baseline (speedup 1.0000x reference)
import jax
import jax.numpy as jnp
from jax.experimental import pallas as pl


def kernel(sparse_rows, sparse_cols, sparse_vals, dense_feat, W, b):
    raise NotImplementedError("write your pallas kernel here")


# speedup vs baseline: 13.4763x; 13.4763x over previous
import functools

import jax
import jax.numpy as jnp
from jax import lax
from jax.experimental import pallas as pl
from jax.experimental.pallas import tpu as pltpu
from jax.experimental.pallas import tpu_sc as plsc

BATCH = 16384
VOCAB = 100000
D = 64
DU = 128
NNZ = 819200
NC = 2
NS = 16
LANES = 16
NW = NC * NS
PER_W = NNZ // NW
SUB = 128
N_SUBS = PER_W // SUB
NBUF = 4
OUTER = 1024
N_SUB = OUTER // SUB
N_OUTER = PER_W // OUTER
ROWS_PER_TILE = BATCH // NS


def _sc_partial(rows2d, cols2d, vals, w):
  mesh = plsc.VectorSubcoreMesh(core_axis_name="c", subcore_axis_name="s")

  @functools.partial(
      pl.kernel,
      out_type=jax.ShapeDtypeStruct((NC, BATCH, D), jnp.float32),
      mesh=mesh,
      scratch_types=[
          pltpu.VMEM((3, N_SUB, SUB), jnp.int32),
          pltpu.VMEM((3, N_SUB, SUB), jnp.int32),
          pltpu.VMEM((3, OUTER), jnp.float32),
          pltpu.VMEM((NBUF, SUB, D), jnp.float32),
          pltpu.VMEM_SHARED((BATCH, D), jnp.float32),
          [pltpu.SemaphoreType.DMA] * NBUF,
          [pltpu.SemaphoreType.DMA] * NBUF,
          pltpu.SemaphoreType.DMA,
      ],
      compiler_params=pltpu.CompilerParams(use_tc_tiling_on_sc=False),
  )
  def k(rows_hbm, cols_hbm, vals_hbm, w_hbm, out_hbm,
        cols_v, rows_v, vals_v, gath_v, accum, gsems, ssems, stsem):
    c = lax.axis_index("c")
    s = lax.axis_index("s")
    wid = c * NS + s

    def stage(blk):
      h = lax.rem(blk, 3)
      c0 = pl.multiple_of(wid * N_SUBS + blk * N_SUB, 8)
      v0 = pl.multiple_of(wid * PER_W + blk * OUTER, 8)
      pltpu.async_copy(cols_hbm.at[pl.ds(c0, N_SUB)], cols_v.at[h], stsem)
      pltpu.async_copy(rows_hbm.at[pl.ds(c0, N_SUB)], rows_v.at[h], stsem)
      pltpu.async_copy(vals_hbm.at[pl.ds(v0, OUTER)], vals_v.at[h], stsem)

    def wait_stage():
      pltpu.make_async_copy(cols_hbm.at[pl.ds(0, N_SUB)], cols_v.at[0],
                            stsem).wait()
      pltpu.make_async_copy(rows_hbm.at[pl.ds(0, N_SUB)], rows_v.at[0],
                            stsem).wait()
      pltpu.make_async_copy(vals_hbm.at[pl.ds(0, OUTER)], vals_v.at[0],
                            stsem).wait()

    @pl.loop(0, SUB)
    def _(e):
      zero = jnp.zeros((LANES,), jnp.float32)
      for j in range(D // LANES):
        gath_v[0, e, pl.ds(j * LANES, LANES)] = zero

    @pl.loop(0, ROWS_PER_TILE // SUB)
    def _(kk):
      z0 = pl.multiple_of(s * ROWS_PER_TILE + kk * SUB, SUB)
      pltpu.sync_copy(gath_v.at[0], accum.at[pl.ds(z0, SUB)])

    plsc.subcore_barrier()

    def start_gather(h, loc, b):
      pltpu.async_copy(w_hbm.at[cols_v.at[h, loc]], gath_v.at[b], gsems[b])

    def wait_gather(b):
      pltpu.make_async_copy(w_hbm.at[cols_v.at[0, 0]], gath_v.at[b],
                            gsems[b]).wait()

    def start_scatter(h, loc, b):
      pltpu.async_copy(gath_v.at[b], accum.at[rows_v.at[h, loc]], ssems[b],
                       add=True)

    def wait_scatter(b):
      pltpu.make_async_copy(gath_v.at[b], accum.at[rows_v.at[0, 0]],
                            ssems[b]).wait()

    def scale(h, loc, b):
      @pl.loop(0, SUB // LANES)
      def _(e16):
        voff = pl.multiple_of(loc * SUB + e16 * LANES, LANES)
        vchunk = vals_v[h, pl.ds(voff, LANES)]
        for l in range(LANES):
          vs = jnp.full((LANES,), vchunk[l], jnp.float32)
          e = e16 * LANES + l
          for j in range(D // LANES):
            sl = pl.ds(j * LANES, LANES)
            gath_v[b, e, sl] = gath_v[b, e, sl] * vs

    stage(jnp.int32(0))
    wait_stage()
    stage(jnp.int32(1))
    start_gather(0, 0, 0)
    start_gather(0, 1, 1)

    @pl.loop(0, N_OUTER)
    def _(blk):
      h_cur = lax.rem(blk, 3)
      h_nxt = lax.rem(blk + 1, 3)
      for bl in range(N_SUB):
        t_is_tail = bl >= N_SUB - 2
        b = bl % NBUF
        bn = (b + 2) % NBUF

        if bl < 2:
          @pl.when(blk > 0)
          def _():
            wait_scatter(bn)
        else:
          wait_scatter(bn)

        if t_is_tail:
          if bl == N_SUB - 2:
            @pl.when(blk < N_OUTER - 1)
            def _():
              wait_stage()

          @pl.when(blk < N_OUTER - 1)
          def _():
            start_gather(h_nxt, bl + 2 - N_SUB, bn)
        else:
          start_gather(h_cur, bl + 2, bn)

        wait_gather(b)
        if bl == N_SUB - 1:
          @pl.when(blk < N_OUTER - 2)
          def _():
            stage(blk + 2)

        scale(h_cur, bl, b)
        start_scatter(h_cur, bl, b)

    wait_scatter((N_SUBS - 2) % NBUF)
    wait_scatter((N_SUBS - 1) % NBUF)

    plsc.subcore_barrier()
    o0 = pl.multiple_of(s * ROWS_PER_TILE, ROWS_PER_TILE)
    pltpu.sync_copy(accum.at[pl.ds(o0, ROWS_PER_TILE)],
                    out_hbm.at[c, pl.ds(o0, ROWS_PER_TILE)])

  return k(rows2d, cols2d, vals, w)


def _combine(partial, dense_feat, b2d):
  r = 512

  def body(p_ref, d_ref, b_ref, o_ref):
    sd = p_ref[0] + p_ref[1] + b_ref[...]
    o_ref[...] = jnp.concatenate([sd, d_ref[...]], axis=-1)

  return pl.pallas_call(
      body,
      grid=(BATCH // r,),
      in_specs=[
          pl.BlockSpec((NC, r, D), lambda i: (0, i, 0)),
          pl.BlockSpec((r, DU), lambda i: (i, 0)),
          pl.BlockSpec((1, D), lambda i: (0, 0)),
      ],
      out_specs=pl.BlockSpec((r, D + DU), lambda i: (i, 0)),
      out_shape=jax.ShapeDtypeStruct((BATCH, D + DU), jnp.float32),
  )(partial, dense_feat, b2d)


def kernel(sparse_rows, sparse_cols, sparse_vals, dense_feat, W, b):
  rows2d = sparse_rows.astype(jnp.int32).reshape(NNZ // SUB, SUB)
  cols2d = sparse_cols.astype(jnp.int32).reshape(NNZ // SUB, SUB)
  partial = _sc_partial(rows2d, cols2d, sparse_vals, W)
  return _combine(partial, dense_feat, b.reshape(1, D))


# speedup vs baseline: 17.4075x; 1.2917x over previous
import functools

import jax
import jax.numpy as jnp
from jax import lax
from jax.experimental import pallas as pl
from jax.experimental.pallas import tpu as pltpu
from jax.experimental.pallas import tpu_sc as plsc

BATCH = 16384
VOCAB = 100000
D = 64
DU = 128
NNZ = 819200
NC = 2
NS = 16
LANES = 16
NW = NC * NS
PER_W = NNZ // NW
SUB = 128
N_SUBS = PER_W // SUB
NBUF = 4
OUTER = 1024
N_SUB = OUTER // SUB
N_OUTER = PER_W // OUTER
ROWS_PER_TILE = BATCH // NS


def _sc_partial(rows2d, cols2d, vals, w):
  mesh = plsc.VectorSubcoreMesh(core_axis_name="c", subcore_axis_name="s")

  @functools.partial(
      pl.kernel,
      out_type=jax.ShapeDtypeStruct((NC, BATCH, D), jnp.float32),
      mesh=mesh,
      scratch_types=[
          pltpu.VMEM((3, N_SUB, SUB), jnp.int32),
          pltpu.VMEM((3, N_SUB, SUB), jnp.int32),
          pltpu.VMEM((3, OUTER), jnp.float32),
          pltpu.VMEM((NBUF, SUB, D), jnp.float32),
          pltpu.VMEM_SHARED((BATCH, D), jnp.float32),
          [pltpu.SemaphoreType.DMA] * NBUF,
          [pltpu.SemaphoreType.DMA] * NBUF,
          pltpu.SemaphoreType.DMA,
      ],
      compiler_params=pltpu.CompilerParams(use_tc_tiling_on_sc=False),
  )
  def k(rows_hbm, cols_hbm, vals_hbm, w_hbm, out_hbm,
        cols_v, rows_v, vals_v, gath_v, accum, gsems, ssems, stsem):
    c = lax.axis_index("c")
    s = lax.axis_index("s")
    wid = c * NS + s

    def stage(blk):
      h = lax.rem(blk, 3)
      c0 = pl.multiple_of(wid * N_SUBS + blk * N_SUB, 8)
      v0 = pl.multiple_of(wid * PER_W + blk * OUTER, 8)
      pltpu.async_copy(cols_hbm.at[pl.ds(c0, N_SUB)], cols_v.at[h], stsem)
      pltpu.async_copy(rows_hbm.at[pl.ds(c0, N_SUB)], rows_v.at[h], stsem)
      pltpu.async_copy(vals_hbm.at[pl.ds(v0, OUTER)], vals_v.at[h], stsem)

    def wait_stage():
      pltpu.make_async_copy(cols_hbm.at[pl.ds(0, N_SUB)], cols_v.at[0],
                            stsem).wait()
      pltpu.make_async_copy(rows_hbm.at[pl.ds(0, N_SUB)], rows_v.at[0],
                            stsem).wait()
      pltpu.make_async_copy(vals_hbm.at[pl.ds(0, OUTER)], vals_v.at[0],
                            stsem).wait()

    @pl.loop(0, SUB)
    def _(e):
      zero = jnp.zeros((LANES,), jnp.float32)
      for j in range(D // LANES):
        gath_v[0, e, pl.ds(j * LANES, LANES)] = zero

    @pl.loop(0, ROWS_PER_TILE // SUB)
    def _(kk):
      z0 = pl.multiple_of(s * ROWS_PER_TILE + kk * SUB, SUB)
      pltpu.sync_copy(gath_v.at[0], accum.at[pl.ds(z0, SUB)])

    plsc.subcore_barrier()

    def start_gather(h, loc, b):
      pltpu.async_copy(w_hbm.at[cols_v.at[h, loc]], gath_v.at[b], gsems[b])

    def wait_gather(b):
      pltpu.make_async_copy(w_hbm.at[cols_v.at[0, 0]], gath_v.at[b],
                            gsems[b]).wait()

    def start_scatter(h, loc, b):
      pltpu.async_copy(gath_v.at[b], accum.at[rows_v.at[h, loc]], ssems[b],
                       add=True)

    def wait_scatter(b):
      pltpu.make_async_copy(gath_v.at[b], accum.at[rows_v.at[0, 0]],
                            ssems[b]).wait()

    def scale(h, loc, b):
      @plsc.parallel_loop(0, SUB // LANES, 1, unroll=2)
      def _(e16):
        voff = pl.multiple_of(loc * SUB + e16 * LANES, LANES)
        vchunk = vals_v[h, pl.ds(voff, LANES)]
        for l in range(LANES):
          vs = jnp.full((LANES,), vchunk[l], jnp.float32)
          e = e16 * LANES + l
          for j in range(D // LANES):
            sl = pl.ds(j * LANES, LANES)
            gath_v[b, e, sl] = gath_v[b, e, sl] * vs

    stage(jnp.int32(0))
    wait_stage()
    stage(jnp.int32(1))
    start_gather(0, 0, 0)
    start_gather(0, 1, 1)

    @pl.loop(0, N_OUTER)
    def _(blk):
      h_cur = lax.rem(blk, 3)
      h_nxt = lax.rem(blk + 1, 3)
      for bl in range(N_SUB):
        t_is_tail = bl >= N_SUB - 2
        b = bl % NBUF
        bn = (b + 2) % NBUF

        if bl < 2:
          @pl.when(blk > 0)
          def _():
            wait_scatter(bn)
        else:
          wait_scatter(bn)

        if t_is_tail:
          if bl == N_SUB - 2:
            @pl.when(blk < N_OUTER - 1)
            def _():
              wait_stage()

          @pl.when(blk < N_OUTER - 1)
          def _():
            start_gather(h_nxt, bl + 2 - N_SUB, bn)
        else:
          start_gather(h_cur, bl + 2, bn)

        wait_gather(b)
        if bl == N_SUB - 1:
          @pl.when(blk < N_OUTER - 2)
          def _():
            stage(blk + 2)

        scale(h_cur, bl, b)
        start_scatter(h_cur, bl, b)

    wait_scatter((N_SUBS - 2) % NBUF)
    wait_scatter((N_SUBS - 1) % NBUF)

    plsc.subcore_barrier()
    o0 = pl.multiple_of(s * ROWS_PER_TILE, ROWS_PER_TILE)
    pltpu.sync_copy(accum.at[pl.ds(o0, ROWS_PER_TILE)],
                    out_hbm.at[c, pl.ds(o0, ROWS_PER_TILE)])

  return k(rows2d, cols2d, vals, w)


def _combine(partial, dense_feat, b2d):
  r = 512

  def body(p_ref, d_ref, b_ref, o_ref):
    sd = p_ref[0] + p_ref[1] + b_ref[...]
    o_ref[...] = jnp.concatenate([sd, d_ref[...]], axis=-1)

  return pl.pallas_call(
      body,
      grid=(BATCH // r,),
      in_specs=[
          pl.BlockSpec((NC, r, D), lambda i: (0, i, 0)),
          pl.BlockSpec((r, DU), lambda i: (i, 0)),
          pl.BlockSpec((1, D), lambda i: (0, 0)),
      ],
      out_specs=pl.BlockSpec((r, D + DU), lambda i: (i, 0)),
      out_shape=jax.ShapeDtypeStruct((BATCH, D + DU), jnp.float32),
  )(partial, dense_feat, b2d)


def kernel(sparse_rows, sparse_cols, sparse_vals, dense_feat, W, b):
  rows2d = sparse_rows.astype(jnp.int32).reshape(NNZ // SUB, SUB)
  cols2d = sparse_cols.astype(jnp.int32).reshape(NNZ // SUB, SUB)
  partial = _sc_partial(rows2d, cols2d, sparse_vals, W)
  return _combine(partial, dense_feat, b.reshape(1, D))


# speedup vs baseline: 18.2823x; 1.0503x over previous
import functools

import jax
import jax.numpy as jnp
from jax import lax
from jax.experimental import pallas as pl
from jax.experimental.pallas import tpu as pltpu
from jax.experimental.pallas import tpu_sc as plsc

BATCH = 16384
VOCAB = 100000
D = 64
DU = 128
NNZ = 819200
NC = 2
NS = 16
LANES = 16
NW = NC * NS
PER_W = NNZ // NW
SUB = 128
N_SUBS = PER_W // SUB
NBUF = 4
OUTER = 1024
N_SUB = OUTER // SUB
N_OUTER = PER_W // OUTER
ROWS_PER_TILE = BATCH // NS


def _sc_partial(rows2d, cols2d, vals, w):
  mesh = plsc.VectorSubcoreMesh(core_axis_name="c", subcore_axis_name="s")

  @functools.partial(
      pl.kernel,
      out_type=jax.ShapeDtypeStruct((NC, BATCH, D), jnp.float32),
      mesh=mesh,
      scratch_types=[
          pltpu.VMEM((3, N_SUB, SUB), jnp.int32),
          pltpu.VMEM((3, N_SUB, SUB), jnp.int32),
          pltpu.VMEM((3, OUTER), jnp.float32),
          pltpu.VMEM((NBUF, SUB, D), jnp.float32),
          pltpu.VMEM_SHARED((BATCH, D), jnp.float32),
          [pltpu.SemaphoreType.DMA] * NBUF,
          [pltpu.SemaphoreType.DMA] * NBUF,
          pltpu.SemaphoreType.DMA,
      ],
      compiler_params=pltpu.CompilerParams(use_tc_tiling_on_sc=False),
  )
  def k(rows_hbm, cols_hbm, vals_hbm, w_hbm, out_hbm,
        cols_v, rows_v, vals_v, gath_v, accum, gsems, ssems, stsem):
    c = lax.axis_index("c")
    s = lax.axis_index("s")
    wid = c * NS + s

    def stage(blk):
      h = lax.rem(blk, 3)
      c0 = pl.multiple_of(wid * N_SUBS + blk * N_SUB, 8)
      v0 = pl.multiple_of(wid * PER_W + blk * OUTER, 8)
      pltpu.async_copy(cols_hbm.at[pl.ds(c0, N_SUB)], cols_v.at[h], stsem)
      pltpu.async_copy(rows_hbm.at[pl.ds(c0, N_SUB)], rows_v.at[h], stsem)
      pltpu.async_copy(vals_hbm.at[pl.ds(v0, OUTER)], vals_v.at[h], stsem)

    def wait_stage():
      pltpu.make_async_copy(cols_hbm.at[pl.ds(0, N_SUB)], cols_v.at[0],
                            stsem).wait()
      pltpu.make_async_copy(rows_hbm.at[pl.ds(0, N_SUB)], rows_v.at[0],
                            stsem).wait()
      pltpu.make_async_copy(vals_hbm.at[pl.ds(0, OUTER)], vals_v.at[0],
                            stsem).wait()

    @pl.loop(0, SUB)
    def _(e):
      zero = jnp.zeros((LANES,), jnp.float32)
      for j in range(D // LANES):
        gath_v[0, e, pl.ds(j * LANES, LANES)] = zero

    @pl.loop(0, ROWS_PER_TILE // SUB)
    def _(kk):
      z0 = pl.multiple_of(s * ROWS_PER_TILE + kk * SUB, SUB)
      pltpu.sync_copy(gath_v.at[0], accum.at[pl.ds(z0, SUB)])

    plsc.subcore_barrier()

    def start_gather(h, loc, b):
      pltpu.async_copy(w_hbm.at[cols_v.at[h, loc]], gath_v.at[b], gsems[b])

    def wait_gather(b):
      pltpu.make_async_copy(w_hbm.at[cols_v.at[0, 0]], gath_v.at[b],
                            gsems[b]).wait()

    def start_scatter(h, loc, b):
      pltpu.async_copy(gath_v.at[b], accum.at[rows_v.at[h, loc]], ssems[b],
                       add=True)

    def wait_scatter(b):
      pltpu.make_async_copy(gath_v.at[b], accum.at[rows_v.at[0, 0]],
                            ssems[b]).wait()

    def scale(h, loc, b):
      @plsc.parallel_loop(0, SUB // LANES, 1, unroll=4)
      def _(e16):
        voff = pl.multiple_of(loc * SUB + e16 * LANES, LANES)
        vchunk = vals_v[h, pl.ds(voff, LANES)]
        for l in range(LANES):
          vs = jnp.full((LANES,), vchunk[l], jnp.float32)
          e = e16 * LANES + l
          for j in range(D // LANES):
            sl = pl.ds(j * LANES, LANES)
            gath_v[b, e, sl] = gath_v[b, e, sl] * vs

    stage(jnp.int32(0))
    wait_stage()
    stage(jnp.int32(1))
    start_gather(0, 0, 0)
    start_gather(0, 1, 1)

    @pl.loop(0, N_OUTER)
    def _(blk):
      h_cur = lax.rem(blk, 3)
      h_nxt = lax.rem(blk + 1, 3)
      for bl in range(N_SUB):
        t_is_tail = bl >= N_SUB - 2
        b = bl % NBUF
        bn = (b + 2) % NBUF

        if bl < 2:
          @pl.when(blk > 0)
          def _():
            wait_scatter(bn)
        else:
          wait_scatter(bn)

        if t_is_tail:
          if bl == N_SUB - 2:
            @pl.when(blk < N_OUTER - 1)
            def _():
              wait_stage()

          @pl.when(blk < N_OUTER - 1)
          def _():
            start_gather(h_nxt, bl + 2 - N_SUB, bn)
        else:
          start_gather(h_cur, bl + 2, bn)

        wait_gather(b)
        if bl == N_SUB - 1:
          @pl.when(blk < N_OUTER - 2)
          def _():
            stage(blk + 2)

        scale(h_cur, bl, b)
        start_scatter(h_cur, bl, b)

    wait_scatter((N_SUBS - 2) % NBUF)
    wait_scatter((N_SUBS - 1) % NBUF)

    plsc.subcore_barrier()
    o0 = pl.multiple_of(s * ROWS_PER_TILE, ROWS_PER_TILE)
    pltpu.sync_copy(accum.at[pl.ds(o0, ROWS_PER_TILE)],
                    out_hbm.at[c, pl.ds(o0, ROWS_PER_TILE)])

  return k(rows2d, cols2d, vals, w)


def _combine(partial, dense_feat, b2d):
  r = 512

  def body(p_ref, d_ref, b_ref, o_ref):
    sd = p_ref[0] + p_ref[1] + b_ref[...]
    o_ref[...] = jnp.concatenate([sd, d_ref[...]], axis=-1)

  return pl.pallas_call(
      body,
      grid=(BATCH // r,),
      in_specs=[
          pl.BlockSpec((NC, r, D), lambda i: (0, i, 0)),
          pl.BlockSpec((r, DU), lambda i: (i, 0)),
          pl.BlockSpec((1, D), lambda i: (0, 0)),
      ],
      out_specs=pl.BlockSpec((r, D + DU), lambda i: (i, 0)),
      out_shape=jax.ShapeDtypeStruct((BATCH, D + DU), jnp.float32),
  )(partial, dense_feat, b2d)


def kernel(sparse_rows, sparse_cols, sparse_vals, dense_feat, W, b):
  rows2d = sparse_rows.astype(jnp.int32).reshape(NNZ // SUB, SUB)
  cols2d = sparse_cols.astype(jnp.int32).reshape(NNZ // SUB, SUB)
  partial = _sc_partial(rows2d, cols2d, sparse_vals, W)
  return _combine(partial, dense_feat, b.reshape(1, D))


# speedup vs baseline: 18.4641x; 1.0099x over previous
import functools

import jax
import jax.numpy as jnp
from jax import lax
from jax.experimental import pallas as pl
from jax.experimental.pallas import tpu as pltpu
from jax.experimental.pallas import tpu_sc as plsc

BATCH = 16384
VOCAB = 100000
D = 64
DU = 128
NNZ = 819200
NC = 2
NS = 16
LANES = 16
NW = NC * NS
PER_W = NNZ // NW
SUB = 128
N_SUBS = PER_W // SUB
NBUF = 5
LEAD = 3
OUTER = 1024
N_SUB = OUTER // SUB
N_OUTER = PER_W // OUTER
ROWS_PER_TILE = BATCH // NS


def _sc_partial(rows2d, cols2d, vals, w):
  mesh = plsc.VectorSubcoreMesh(core_axis_name="c", subcore_axis_name="s")

  @functools.partial(
      pl.kernel,
      out_type=jax.ShapeDtypeStruct((NC, BATCH, D), jnp.float32),
      mesh=mesh,
      scratch_types=[
          pltpu.VMEM((3, N_SUB, SUB), jnp.int32),
          pltpu.VMEM((3, N_SUB, SUB), jnp.int32),
          pltpu.VMEM((3, OUTER), jnp.float32),
          pltpu.VMEM((NBUF, SUB, D), jnp.float32),
          pltpu.VMEM_SHARED((BATCH, D), jnp.float32),
          [pltpu.SemaphoreType.DMA] * NBUF,
          [pltpu.SemaphoreType.DMA] * NBUF,
          pltpu.SemaphoreType.DMA,
      ],
      compiler_params=pltpu.CompilerParams(use_tc_tiling_on_sc=False),
  )
  def k(rows_hbm, cols_hbm, vals_hbm, w_hbm, out_hbm,
        cols_v, rows_v, vals_v, gath_v, accum, gsems, ssems, stsem):
    c = lax.axis_index("c")
    s = lax.axis_index("s")
    wid = c * NS + s

    def stage(blk):
      h = lax.rem(blk, 3)
      c0 = pl.multiple_of(wid * N_SUBS + blk * N_SUB, 8)
      v0 = pl.multiple_of(wid * PER_W + blk * OUTER, 8)
      pltpu.async_copy(cols_hbm.at[pl.ds(c0, N_SUB)], cols_v.at[h], stsem)
      pltpu.async_copy(rows_hbm.at[pl.ds(c0, N_SUB)], rows_v.at[h], stsem)
      pltpu.async_copy(vals_hbm.at[pl.ds(v0, OUTER)], vals_v.at[h], stsem)

    def wait_stage():
      pltpu.make_async_copy(cols_hbm.at[pl.ds(0, N_SUB)], cols_v.at[0],
                            stsem).wait()
      pltpu.make_async_copy(rows_hbm.at[pl.ds(0, N_SUB)], rows_v.at[0],
                            stsem).wait()
      pltpu.make_async_copy(vals_hbm.at[pl.ds(0, OUTER)], vals_v.at[0],
                            stsem).wait()

    @pl.loop(0, SUB)
    def _(e):
      zero = jnp.zeros((LANES,), jnp.float32)
      for j in range(D // LANES):
        gath_v[0, e, pl.ds(j * LANES, LANES)] = zero

    @pl.loop(0, ROWS_PER_TILE // SUB)
    def _(kk):
      z0 = pl.multiple_of(s * ROWS_PER_TILE + kk * SUB, SUB)
      pltpu.sync_copy(gath_v.at[0], accum.at[pl.ds(z0, SUB)])

    plsc.subcore_barrier()

    def start_gather(h, loc, b):
      pltpu.async_copy(w_hbm.at[cols_v.at[h, loc]], gath_v.at[b], gsems[b])

    def wait_gather(b):
      pltpu.make_async_copy(w_hbm.at[cols_v.at[0, 0]], gath_v.at[b],
                            gsems[b]).wait()

    def start_scatter(h, loc, b):
      pltpu.async_copy(gath_v.at[b], accum.at[rows_v.at[h, loc]], ssems[b],
                       add=True)

    def wait_scatter(b):
      pltpu.make_async_copy(gath_v.at[b], accum.at[rows_v.at[0, 0]],
                            ssems[b]).wait()

    def scale(h, loc, b):
      @plsc.parallel_loop(0, SUB // LANES, 1, unroll=4)
      def _(e16):
        voff = pl.multiple_of(loc * SUB + e16 * LANES, LANES)
        vchunk = vals_v[h, pl.ds(voff, LANES)]
        for l in range(LANES):
          vs = jnp.full((LANES,), vchunk[l], jnp.float32)
          e = e16 * LANES + l
          for j in range(D // LANES):
            sl = pl.ds(j * LANES, LANES)
            gath_v[b, e, sl] = gath_v[b, e, sl] * vs

    stage(jnp.int32(0))
    wait_stage()
    stage(jnp.int32(1))
    for u in range(LEAD):
      start_gather(0, u, u)

    @pl.loop(0, N_SUBS // NBUF)
    def _(i):
      for g in range(NBUF):
        t = i * NBUF + g
        b = g
        bn = (g + LEAD) % NBUF
        blk = t // N_SUB
        loc = lax.rem(t, N_SUB)
        h_cur = lax.rem(blk, 3)
        u = t + LEAD
        ublk = u // N_SUB
        uloc = lax.rem(u, N_SUB)
        h_u = lax.rem(ublk, 3)

        if g < 2:
          @pl.when(i > 0)
          def _():
            wait_scatter(bn)
        else:
          wait_scatter(bn)

        @pl.when(jnp.logical_and(lax.rem(t, N_SUB) == N_SUB - LEAD,
                                 blk < N_OUTER - 1))
        def _():
          wait_stage()

        @pl.when(u < N_SUBS)
        def _():
          start_gather(h_u, uloc, bn)

        wait_gather(b)

        @pl.when(jnp.logical_and(lax.rem(t, N_SUB) == N_SUB - 1,
                                 blk < N_OUTER - 2))
        def _():
          stage(blk + 2)

        scale(h_cur, loc, b)
        start_scatter(h_cur, loc, b)

    for u in range(N_SUBS - LEAD + 1, N_SUBS):
      wait_scatter(u % NBUF)

    plsc.subcore_barrier()
    o0 = pl.multiple_of(s * ROWS_PER_TILE, ROWS_PER_TILE)
    pltpu.sync_copy(accum.at[pl.ds(o0, ROWS_PER_TILE)],
                    out_hbm.at[c, pl.ds(o0, ROWS_PER_TILE)])

  return k(rows2d, cols2d, vals, w)


def _combine(partial, dense_feat, b2d):
  r = 512

  def body(p_ref, d_ref, b_ref, o_ref):
    sd = p_ref[0] + p_ref[1] + b_ref[...]
    o_ref[...] = jnp.concatenate([sd, d_ref[...]], axis=-1)

  return pl.pallas_call(
      body,
      grid=(BATCH // r,),
      in_specs=[
          pl.BlockSpec((NC, r, D), lambda i: (0, i, 0)),
          pl.BlockSpec((r, DU), lambda i: (i, 0)),
          pl.BlockSpec((1, D), lambda i: (0, 0)),
      ],
      out_specs=pl.BlockSpec((r, D + DU), lambda i: (i, 0)),
      out_shape=jax.ShapeDtypeStruct((BATCH, D + DU), jnp.float32),
  )(partial, dense_feat, b2d)


def kernel(sparse_rows, sparse_cols, sparse_vals, dense_feat, W, b):
  rows2d = sparse_rows.astype(jnp.int32).reshape(NNZ // SUB, SUB)
  cols2d = sparse_cols.astype(jnp.int32).reshape(NNZ // SUB, SUB)
  partial = _sc_partial(rows2d, cols2d, sparse_vals, W)
  return _combine(partial, dense_feat, b.reshape(1, D))
